# Initial kernel scaffold; baseline (speedup 1.0000x reference)
import jax
import jax.numpy as jnp
from jax.experimental import pallas as pl


def kernel(beta, embed, slice_id, is_cp):
    raise NotImplementedError("write your pallas kernel here")


# speedup vs baseline: 9.0481x; 9.0481x over previous
import functools

import jax
import jax.numpy as jnp
from jax import lax
from jax.experimental import pallas as pl
from jax.experimental.pallas import tpu as pltpu

_S = 128
_BJ = 512


def _softplus(x):
    return jnp.maximum(x, 0.0) + jnp.log(1.0 + jnp.exp(-jnp.abs(x)))


def _body(beta_ref, e_ref, sid_ref, cp_ref, out_ref, acc_ref, *, n, b_total):
    b = pl.program_id(0)

    @pl.when(b == 0)
    def _init():
        acc_ref[0] = 0.0
        acc_ref[1] = 0.0

    f32 = jnp.float32
    E = e_ref[0]
    bb = beta_ref[0]
    w = cp_ref[0]
    sid = sid_ref[0]

    n_cp = jnp.sum(w)
    n_non = n - n_cp

    pos = jnp.sum(_softplus(-bb) * w) / jnp.maximum(n_cp, 1.0)
    neg_sum = jnp.sum(_softplus(bb) * (1.0 - w))
    neg = jnp.where(n_non > 0, neg_sum / jnp.maximum(n_non, 1.0), 0.0)
    beta_loss = pos + 0.5 * neg

    seg_i = lax.broadcasted_iota(jnp.int32, (_S, n), 0)
    ohT = (seg_i == sid).astype(f32)
    cnt = jnp.sum(ohT, axis=1, keepdims=True)
    dn = (((1,), (0,)), ((), ()))
    sum_e = lax.dot_general(ohT, E, dn, preferred_element_type=f32)
    E2 = E * E
    sq_col = jnp.sum(E2, axis=1, keepdims=True)
    sumsq = lax.dot_general(ohT, sq_col, dn, preferred_element_type=f32)

    idx_row = lax.broadcasted_iota(jnp.int32, (1, n), 1)
    cand = jnp.where(w > 0, idx_row, n)
    m = jnp.where(ohT > 0, cand, n)
    first_cp = jnp.min(m, axis=1, keepdims=True)
    seg_j = lax.broadcasted_iota(jnp.int32, (_S, n), 1)
    ohF = (seg_j == first_cp).astype(f32)
    C = lax.dot_general(ohF, E, dn, preferred_element_type=f32)
    dot_cs = jnp.sum(C * sum_e, axis=1, keepdims=True)
    csq = jnp.sum(C * C, axis=1, keepdims=True)
    inst_mean = (sumsq - 2.0 * dot_cs + cnt * csq) / jnp.maximum(cnt, 1.0)
    use = (cnt > 0) & (first_cp < n)
    attraction = jnp.sum(jnp.where(use, inst_mean, 0.0))

    dn_bt = (((1,), (1,)), ((), ()))
    ones_d = jnp.ones((1, E.shape[1]), f32)
    rep = 0.0
    for jb in range(n // _BJ):
        Ej = E[jb * _BJ:(jb + 1) * _BJ, :]
        G = lax.dot_general(E, Ej, dn_bt, preferred_element_type=f32)
        sqj_row = lax.dot_general(ones_d, E2[jb * _BJ:(jb + 1) * _BJ, :],
                                  dn_bt, preferred_element_type=f32)
        ex = jnp.exp(2.0 * G - sq_col - sqj_row)
        t = lax.dot_general(w, ex, dn, preferred_element_type=f32)
        rep = rep + jnp.sum(t * w[:, jb * _BJ:(jb + 1) * _BJ])
    rep_mean = rep / jnp.maximum(n_cp * n_cp, 1.0)
    repulsion = jnp.where(n_cp > 1, rep_mean, 0.0)

    active = n_cp > 0
    contrib = beta_loss + attraction + repulsion
    acc_ref[0] += jnp.where(active, contrib, 0.0)
    acc_ref[1] += jnp.where(active, 1.0, 0.0)
    total = acc_ref[0]
    countf = acc_ref[1]
    loss = jnp.where(countf > 0.0, total / jnp.maximum(countf, 1.0), 0.0)
    out_ref[...] = jnp.broadcast_to(loss, (1, 1))


@functools.partial(jax.jit, static_argnames=())
def kernel(beta, embed, slice_id, is_cp):
    B, N, D = embed.shape
    beta_row = beta[..., 0].astype(jnp.float32).reshape(B, 1, N)
    cp_row = is_cp.astype(jnp.float32).reshape(B, 1, N)
    sid_row = slice_id.astype(jnp.int32).reshape(B, 1, N)

    out = pl.pallas_call(
        functools.partial(_body, n=N, b_total=B),
        grid=(B,),
        in_specs=[
            pl.BlockSpec((1, 1, N), lambda b: (b, 0, 0)),
            pl.BlockSpec((1, N, D), lambda b: (b, 0, 0)),
            pl.BlockSpec((1, 1, N), lambda b: (b, 0, 0)),
            pl.BlockSpec((1, 1, N), lambda b: (b, 0, 0)),
        ],
        out_specs=pl.BlockSpec((1, 1), lambda b: (0, 0)),
        out_shape=jax.ShapeDtypeStruct((1, 1), jnp.float32),
        scratch_shapes=[pltpu.SMEM((2,), jnp.float32)],
    )(beta_row, embed, sid_row, cp_row)
    return out[0, 0]


# speedup vs baseline: 18.1984x; 2.0113x over previous
import functools

import jax
import jax.numpy as jnp
from jax import lax
from jax.experimental import pallas as pl
from jax.experimental.pallas import tpu as pltpu

_S = 128
_BJ = 512


def _softplus(x):
    return jnp.maximum(x, 0.0) + jnp.log(1.0 + jnp.exp(-jnp.abs(x)))


def _body(beta_ref, e_ref, sid_ref, cp_ref, out_ref, acc_ref, *, n, b_total):
    b = pl.program_id(0)

    @pl.when(b == 0)
    def _init():
        acc_ref[0] = 0.0
        acc_ref[1] = 0.0

    f32 = jnp.float32
    E = e_ref[0]
    bb = beta_ref[0]
    w = cp_ref[0]
    sid = sid_ref[0]

    n_cp = jnp.sum(w)
    n_non = n - n_cp

    pos = jnp.sum(_softplus(-bb) * w) / jnp.maximum(n_cp, 1.0)
    neg_sum = jnp.sum(_softplus(bb) * (1.0 - w))
    neg = jnp.where(n_non > 0, neg_sum / jnp.maximum(n_non, 1.0), 0.0)
    beta_loss = pos + 0.5 * neg

    seg_i = lax.broadcasted_iota(jnp.int32, (_S, n), 0)
    ohT = (seg_i == sid).astype(f32)
    cnt = jnp.sum(ohT, axis=1, keepdims=True)
    dn = (((1,), (0,)), ((), ()))
    sum_e = lax.dot_general(ohT, E, dn, preferred_element_type=f32)
    E2 = E * E
    sq_col = jnp.sum(E2, axis=1, keepdims=True)
    sumsq = lax.dot_general(ohT, sq_col, dn, preferred_element_type=f32)

    idx_row = lax.broadcasted_iota(jnp.int32, (1, n), 1)
    cand = jnp.where(w > 0, idx_row, n)
    m = jnp.where(ohT > 0, cand, n)
    first_cp = jnp.min(m, axis=1, keepdims=True)
    seg_j = lax.broadcasted_iota(jnp.int32, (_S, n), 1)
    ohF = (seg_j == first_cp).astype(f32)
    C = lax.dot_general(ohF, E, dn, preferred_element_type=f32)
    dot_cs = jnp.sum(C * sum_e, axis=1, keepdims=True)
    csq = jnp.sum(C * C, axis=1, keepdims=True)
    inst_mean = (sumsq - 2.0 * dot_cs + cnt * csq) / jnp.maximum(cnt, 1.0)
    use = (cnt > 0) & (first_cp < n)
    attraction = jnp.sum(jnp.where(use, inst_mean, 0.0))

    dn_bt = (((1,), (1,)), ((), ()))
    ones_d = jnp.ones((1, E.shape[1]), f32)
    rep = 0.0
    for jb in range(n // _BJ):
        j0 = jb * _BJ
        Ej = E[j0:j0 + _BJ, :]
        sqj_row = lax.dot_general(ones_d, E2[j0:j0 + _BJ, :],
                                  dn_bt, preferred_element_type=f32)
        wj = w[:, j0:j0 + _BJ]
        Gd = lax.dot_general(Ej, Ej, dn_bt, preferred_element_type=f32)
        exd = jnp.exp(2.0 * Gd - sq_col[j0:j0 + _BJ, :] - sqj_row)
        td = lax.dot_general(wj, exd, dn, preferred_element_type=f32)
        rep = rep + jnp.sum(td * wj)
        if jb > 0:
            Ei = E[:j0, :]
            G = lax.dot_general(Ei, Ej, dn_bt, preferred_element_type=f32)
            ex = jnp.exp(2.0 * G - sq_col[:j0, :] - sqj_row)
            t = lax.dot_general(w[:, :j0], ex, dn, preferred_element_type=f32)
            rep = rep + 2.0 * jnp.sum(t * wj)
    rep_mean = rep / jnp.maximum(n_cp * n_cp, 1.0)
    repulsion = jnp.where(n_cp > 1, rep_mean, 0.0)

    active = n_cp > 0
    contrib = beta_loss + attraction + repulsion
    acc_ref[0] += jnp.where(active, contrib, 0.0)
    acc_ref[1] += jnp.where(active, 1.0, 0.0)
    total = acc_ref[0]
    countf = acc_ref[1]
    loss = jnp.where(countf > 0.0, total / jnp.maximum(countf, 1.0), 0.0)
    out_ref[...] = jnp.broadcast_to(loss, (1, 1))


@functools.partial(jax.jit, static_argnames=())
def kernel(beta, embed, slice_id, is_cp):
    B, N, D = embed.shape
    beta_row = beta[..., 0].astype(jnp.float32).reshape(B, 1, N)
    cp_row = is_cp.astype(jnp.float32).reshape(B, 1, N)
    sid_row = slice_id.astype(jnp.int32).reshape(B, 1, N)

    out = pl.pallas_call(
        functools.partial(_body, n=N, b_total=B),
        grid=(B,),
        in_specs=[
            pl.BlockSpec((1, 1, N), lambda b: (b, 0, 0)),
            pl.BlockSpec((1, N, D), lambda b: (b, 0, 0)),
            pl.BlockSpec((1, 1, N), lambda b: (b, 0, 0)),
            pl.BlockSpec((1, 1, N), lambda b: (b, 0, 0)),
        ],
        out_specs=pl.BlockSpec((1, 1), lambda b: (0, 0)),
        out_shape=jax.ShapeDtypeStruct((1, 1), jnp.float32),
        scratch_shapes=[pltpu.SMEM((2,), jnp.float32)],
    )(beta_row, embed, sid_row, cp_row)
    return out[0, 0]
